# baseline (speedup 1.0000x reference)
import functools

import jax
import jax.numpy as jnp
from jax import lax
from jax.experimental import pallas as pl
from jax.experimental.pallas import tpu as pltpu
from jax.experimental.pallas import tpu_sc as plsc

N = 10000
E = 320000
D = 128
DN = 142
NC = 2
NS = 16
NW = NC * NS
EPW = E // NW
CH = 64
NCH = EPW // CH
TL = EPW - NCH * CH
NP = 10112
RPT = NP // NS


def _proj_body(x_ref, w_ref, o_ref):
    o_ref[...] = jnp.dot(x_ref[...], w_ref[...],
                         preferred_element_type=jnp.float32,
                         precision=lax.Precision.HIGHEST)


def _project(x, w_node):
    return pl.pallas_call(
        _proj_body,
        grid=(5,),
        in_specs=[
            pl.BlockSpec((2000, DN), lambda i: (i, 0)),
            pl.BlockSpec((DN, D), lambda i: (0, 0)),
        ],
        out_specs=pl.BlockSpec((2000, D), lambda i: (i, 0)),
        out_shape=jax.ShapeDtypeStruct((N, D), jnp.float32),
    )(x, w_node)


def _edge_proj_body(at_ref, w_ref, o_ref):
    o_ref[...] = lax.dot_general(
        at_ref[...], w_ref[...], (((0,), (0,)), ((), ())),
        preferred_element_type=jnp.float32,
        precision=lax.Precision.DEFAULT)


def _edge_proj(attr_t8, we8):
    return pl.pallas_call(
        _edge_proj_body,
        grid=(20,),
        in_specs=[
            pl.BlockSpec((8, 16000), lambda i: (0, i)),
            pl.BlockSpec((8, D), lambda i: (0, 0)),
        ],
        out_specs=pl.BlockSpec((16000, D), lambda i: (i, 0)),
        out_shape=jax.ShapeDtypeStruct((E, D), jnp.float32),
    )(attr_t8, we8)


_MESH = plsc.VectorSubcoreMesh(core_axis_name="c", subcore_axis_name="s")


@functools.partial(
    pl.kernel,
    mesh=_MESH,
    out_type=jax.ShapeDtypeStruct((NC, NP, D), jnp.float32),
    scratch_types=[
        pltpu.VMEM((CH,), jnp.int32),
        pltpu.VMEM((CH,), jnp.int32),
        pltpu.VMEM((CH,), jnp.int32),
        pltpu.VMEM((CH,), jnp.int32),
        pltpu.VMEM((CH,), jnp.int32),
        pltpu.VMEM((CH,), jnp.int32),
        pltpu.VMEM((CH, D), jnp.float32),
        pltpu.VMEM((CH, D), jnp.float32),
        pltpu.VMEM((CH, D), jnp.float32),
        pltpu.VMEM((CH, D), jnp.float32),
        pltpu.VMEM((CH, D), jnp.float32),
        pltpu.VMEM((CH, D), jnp.float32),
        pltpu.VMEM((TL,), jnp.int32),
        pltpu.VMEM((TL,), jnp.int32),
        pltpu.VMEM_SHARED((NP, D), jnp.float32),
        pltpu.SemaphoreType.DMA,
        pltpu.SemaphoreType.DMA,
        pltpu.SemaphoreType.DMA,
        pltpu.SemaphoreType.DMA,
        pltpu.SemaphoreType.DMA,
        pltpu.SemaphoreType.DMA,
    ],
)
def _sc_agg(h_hbm, p_hbm, src_hbm, dst_hbm, z128_hbm, agg_out,
            src_0, dst_0, src_1, dst_1, src_2, dst_2,
            rows_0, rows_1, rows_2, p_0, p_1, p_2,
            src_t, dst_t, acc,
            sem_l0, sem_l1, sem_l2, sem_g0, sem_g1, sem_g2):
    cid = lax.axis_index("c")
    sid = lax.axis_index("s")
    wid = cid * NS + sid
    row0 = pl.multiple_of(sid * RPT, 8)
    e0 = pl.multiple_of(wid * EPW, 8)

    SETS = (
        (src_0, dst_0, rows_0, p_0, sem_l0, sem_g0),
        (src_1, dst_1, rows_1, p_1, sem_l1, sem_g1),
        (src_2, dst_2, rows_2, p_2, sem_l2, sem_g2),
    )

    def start_loads(c, st):
        src_x, dst_x, _, p_x, sem, _ = st
        base = pl.multiple_of(e0 + c * CH, 8)
        pltpu.async_copy(src_hbm.at[pl.ds(base, CH)], src_x, sem)
        pltpu.async_copy(dst_hbm.at[pl.ds(base, CH)], dst_x, sem)
        pltpu.async_copy(p_hbm.at[pl.ds(base, CH)], p_x, sem)

    def wait_loads(st):
        src_x, dst_x, _, p_x, sem, _ = st
        pltpu.make_async_copy(src_hbm.at[pl.ds(0, CH)], src_x, sem).wait()
        pltpu.make_async_copy(dst_hbm.at[pl.ds(0, CH)], dst_x, sem).wait()
        pltpu.make_async_copy(p_hbm.at[pl.ds(0, CH)], p_x, sem).wait()

    def start_gather(st):
        src_x, _, rows_x, _, _, semg = st
        pltpu.async_copy(h_hbm.at[src_x], rows_x, semg)

    def wait_gather(st):
        src_x, _, rows_x, _, _, semg = st
        pltpu.make_async_copy(h_hbm.at[src_x], rows_x, semg).wait()

    def scatter(st):
        _, dst_x, rows_x, p_x, _, _ = st
        pltpu.sync_copy(rows_x, acc.at[dst_x], add=True)
        pltpu.sync_copy(p_x, acc.at[dst_x], add=True)

    start_loads(0, SETS[0])
    start_loads(1, SETS[1])
    pltpu.sync_copy(z128_hbm.at[pl.ds(row0, RPT)], acc.at[pl.ds(row0, RPT)])
    plsc.subcore_barrier()

    def body(i, carry):
        for j in range(3):
            c = 3 * i + j
            st = SETS[j]
            prev = SETS[(j + 2) % 3]

            @pl.when(c > 0)
            def _():
                wait_gather(prev)
                scatter(prev)

            @pl.when(c + 2 < NCH)
            def _():
                start_loads(c + 2, prev)

            wait_loads(st)
            start_gather(st)
        return carry

    lax.fori_loop(0, NCH // 3, body, 0)

    last = SETS[(NCH - 1) % 3]
    wait_gather(last)
    scatter(last)

    tbase = pl.multiple_of(e0 + NCH * CH, 8)
    pltpu.sync_copy(src_hbm.at[pl.ds(tbase, TL)], src_t)
    pltpu.sync_copy(dst_hbm.at[pl.ds(tbase, TL)], dst_t)
    pltpu.async_copy(h_hbm.at[src_t], rows_0.at[pl.ds(0, TL)], sem_g0).wait()
    pltpu.sync_copy(p_hbm.at[pl.ds(tbase, TL)], p_0.at[pl.ds(0, TL)])
    pltpu.sync_copy(rows_0.at[pl.ds(0, TL)], acc.at[dst_t], add=True)
    pltpu.sync_copy(p_0.at[pl.ds(0, TL)], acc.at[dst_t], add=True)

    plsc.subcore_barrier()

    pltpu.sync_copy(acc.at[pl.ds(row0, RPT)],
                    agg_out.at[cid, pl.ds(row0, RPT)])


def _fin_body(h_ref, agg_ref, wo_ref, b_ref, o_ref):
    emb = jnp.maximum(h_ref[...] + agg_ref[0] + agg_ref[1], 0.0)
    logit = jnp.dot(emb, wo_ref[...], preferred_element_type=jnp.float32,
                    precision=lax.Precision.HIGHEST) + b_ref[0, 0]
    o_ref[...] = jax.nn.sigmoid(logit)


def _finish(h, agg, w_out, b2):
    return pl.pallas_call(
        _fin_body,
        grid=(5,),
        in_specs=[
            pl.BlockSpec((2000, D), lambda i: (i, 0)),
            pl.BlockSpec((NC, 2000, D), lambda i: (0, i, 0)),
            pl.BlockSpec((D, 1), lambda i: (0, 0)),
            pl.BlockSpec(memory_space=pltpu.SMEM),
        ],
        out_specs=pl.BlockSpec((2000, 1), lambda i: (i, 0)),
        out_shape=jax.ShapeDtypeStruct((N, 1), jnp.float32),
    )(h, agg, w_out, b2)


def kernel(x, edge_index, edge_attr, W_node, W_edge, W_out, b_out):
    ei = edge_index.astype(jnp.int32)
    src = ei[0]
    dst = ei[1]
    z128 = jnp.zeros((NP, D), jnp.float32)

    attr_t8 = jnp.pad(edge_attr.T, ((0, 3), (0, 0)))
    we8 = jnp.pad(W_edge, ((0, 3), (0, 0)))

    h = _project(x, W_node)
    p = _edge_proj(attr_t8, we8)
    agg = _sc_agg(h, p, src, dst, z128)

    return _finish(h, agg, W_out, b_out.reshape(1, 1))[:, 0]


# speedup vs baseline: 1.0969x; 1.0969x over previous
import functools

import jax
import jax.numpy as jnp
from jax import lax
from jax.experimental import pallas as pl
from jax.experimental.pallas import tpu as pltpu
from jax.experimental.pallas import tpu_sc as plsc

N = 10000
E = 320000
D = 128
DN = 142
NC = 2
NS = 16
NW = NC * NS
EPW = E // NW
CH = 64
NCH = EPW // CH
TL = EPW - NCH * CH
NP = 10112
RPT = NP // NS


def _proj_body(x_ref, w_ref, o_ref):
    o_ref[...] = jnp.dot(x_ref[...], w_ref[...],
                         preferred_element_type=jnp.float32,
                         precision=lax.Precision.HIGHEST)


def _project(x, w_node):
    return pl.pallas_call(
        _proj_body,
        grid=(5,),
        in_specs=[
            pl.BlockSpec((2000, DN), lambda i: (i, 0)),
            pl.BlockSpec((DN, D), lambda i: (0, 0)),
        ],
        out_specs=pl.BlockSpec((2000, D), lambda i: (i, 0)),
        out_shape=jax.ShapeDtypeStruct((N, D), jnp.float32),
    )(x, w_node)


def _edge_proj_body(at_ref, w_ref, o_ref):
    o_ref[...] = lax.dot_general(
        at_ref[...], w_ref[...], (((0,), (0,)), ((), ())),
        preferred_element_type=jnp.float32,
        precision=lax.Precision.DEFAULT)


def _edge_proj(attr_t8, we8):
    return pl.pallas_call(
        _edge_proj_body,
        grid=(20,),
        in_specs=[
            pl.BlockSpec((8, 16000), lambda i: (0, i)),
            pl.BlockSpec((8, D), lambda i: (0, 0)),
        ],
        out_specs=pl.BlockSpec((16000, D), lambda i: (i, 0)),
        out_shape=jax.ShapeDtypeStruct((E, D), jnp.float32),
    )(attr_t8, we8)


_MESH = plsc.VectorSubcoreMesh(core_axis_name="c", subcore_axis_name="s")


@functools.partial(
    pl.kernel,
    mesh=_MESH,
    out_type=jax.ShapeDtypeStruct((NC, NP, D), jnp.float32),
    scratch_types=[
        pltpu.VMEM((CH,), jnp.int32),
        pltpu.VMEM((CH,), jnp.int32),
        pltpu.VMEM((CH,), jnp.int32),
        pltpu.VMEM((CH,), jnp.int32),
        pltpu.VMEM((CH, D), jnp.float32),
        pltpu.VMEM((CH, D), jnp.float32),
        pltpu.VMEM((CH, D), jnp.float32),
        pltpu.VMEM((CH, D), jnp.float32),
        pltpu.VMEM((TL,), jnp.int32),
        pltpu.VMEM((TL,), jnp.int32),
        pltpu.VMEM((TL, D), jnp.float32),
        pltpu.VMEM((TL, D), jnp.float32),
        pltpu.VMEM_SHARED((NP, D), jnp.float32),
        pltpu.SemaphoreType.DMA,
        pltpu.SemaphoreType.DMA,
        pltpu.SemaphoreType.DMA,
        pltpu.SemaphoreType.DMA,
    ],
)
def _sc_agg(h_hbm, p_hbm, src_hbm, dst_hbm, z128_hbm, agg_out,
            src_a, dst_a, src_b, dst_b, rows_a, rows_b, p_a, p_b,
            src_t, dst_t, rows_t, p_t, acc,
            sem_la, sem_lb, sem_ga, sem_gb):
    cid = lax.axis_index("c")
    sid = lax.axis_index("s")
    wid = cid * NS + sid
    row0 = pl.multiple_of(sid * RPT, 8)
    e0 = pl.multiple_of(wid * EPW, 8)

    def start_loads(c, src_x, dst_x, p_x, sem):
        base = pl.multiple_of(e0 + c * CH, 8)
        pltpu.async_copy(src_hbm.at[pl.ds(base, CH)], src_x, sem)
        pltpu.async_copy(dst_hbm.at[pl.ds(base, CH)], dst_x, sem)
        pltpu.async_copy(p_hbm.at[pl.ds(base, CH)], p_x, sem)

    def wait_loads(src_x, dst_x, p_x, sem):
        pltpu.make_async_copy(src_hbm.at[pl.ds(0, CH)], src_x, sem).wait()
        pltpu.make_async_copy(dst_hbm.at[pl.ds(0, CH)], dst_x, sem).wait()
        pltpu.make_async_copy(p_hbm.at[pl.ds(0, CH)], p_x, sem).wait()

    def scatter(rows_x, p_x, dst_x):
        pltpu.sync_copy(rows_x, acc.at[dst_x], add=True)
        pltpu.sync_copy(p_x, acc.at[dst_x], add=True)

    start_loads(0, src_a, dst_a, p_a, sem_la)
    pltpu.sync_copy(z128_hbm.at[pl.ds(row0, RPT)], acc.at[pl.ds(row0, RPT)])
    plsc.subcore_barrier()

    def body(i, carry):
        wait_loads(src_a, dst_a, p_a, sem_la)
        pltpu.async_copy(h_hbm.at[src_a], rows_a, sem_ga)

        @pl.when(i > 0)
        def _():
            pltpu.make_async_copy(h_hbm.at[src_b], rows_b, sem_gb).wait()
            scatter(rows_b, p_b, dst_b)

        start_loads(2 * i + 1, src_b, dst_b, p_b, sem_lb)
        wait_loads(src_b, dst_b, p_b, sem_lb)
        pltpu.async_copy(h_hbm.at[src_b], rows_b, sem_gb)

        pltpu.make_async_copy(h_hbm.at[src_a], rows_a, sem_ga).wait()
        scatter(rows_a, p_a, dst_a)

        @pl.when(i < NCH // 2 - 1)
        def _():
            start_loads(2 * i + 2, src_a, dst_a, p_a, sem_la)

        return carry

    lax.fori_loop(0, NCH // 2, body, 0)

    pltpu.make_async_copy(h_hbm.at[src_b], rows_b, sem_gb).wait()
    scatter(rows_b, p_b, dst_b)

    tbase = pl.multiple_of(e0 + NCH * CH, 8)
    pltpu.sync_copy(src_hbm.at[pl.ds(tbase, TL)], src_t)
    pltpu.sync_copy(dst_hbm.at[pl.ds(tbase, TL)], dst_t)
    pltpu.async_copy(h_hbm.at[src_t], rows_t, sem_ga).wait()
    pltpu.sync_copy(p_hbm.at[pl.ds(tbase, TL)], p_t)
    scatter(rows_t, p_t, dst_t)

    plsc.subcore_barrier()

    pltpu.sync_copy(acc.at[pl.ds(row0, RPT)],
                    agg_out.at[cid, pl.ds(row0, RPT)])


def _fin_body(h_ref, agg_ref, wo_ref, b_ref, o_ref):
    emb = jnp.maximum(h_ref[...] + agg_ref[0] + agg_ref[1], 0.0)
    logit = jnp.dot(emb, wo_ref[...], preferred_element_type=jnp.float32,
                    precision=lax.Precision.HIGHEST) + b_ref[0, 0]
    o_ref[...] = jax.nn.sigmoid(logit)


def _finish(h, agg, w_out, b2):
    return pl.pallas_call(
        _fin_body,
        grid=(5,),
        in_specs=[
            pl.BlockSpec((2000, D), lambda i: (i, 0)),
            pl.BlockSpec((NC, 2000, D), lambda i: (0, i, 0)),
            pl.BlockSpec((D, 1), lambda i: (0, 0)),
            pl.BlockSpec(memory_space=pltpu.SMEM),
        ],
        out_specs=pl.BlockSpec((2000, 1), lambda i: (i, 0)),
        out_shape=jax.ShapeDtypeStruct((N, 1), jnp.float32),
    )(h, agg, w_out, b2)


def kernel(x, edge_index, edge_attr, W_node, W_edge, W_out, b_out):
    ei = edge_index.astype(jnp.int32)
    src = ei[0]
    dst = ei[1]
    z128 = jnp.zeros((NP, D), jnp.float32)

    attr_t8 = jnp.pad(edge_attr.T, ((0, 3), (0, 0)))
    we8 = jnp.pad(W_edge, ((0, 3), (0, 0)))

    h = _project(x, W_node)
    p = _edge_proj(attr_t8, we8)
    agg = _sc_agg(h, p, src, dst, z128)

    return _finish(h, agg, W_out, b_out.reshape(1, 1)).reshape(N)


# speedup vs baseline: 1.1636x; 1.0608x over previous
import functools

import jax
import jax.numpy as jnp
from jax import lax
from jax.experimental import pallas as pl
from jax.experimental.pallas import tpu as pltpu
from jax.experimental.pallas import tpu_sc as plsc

N = 10000
E = 320000
D = 128
DN = 142
NC = 2
NS = 16
NW = NC * NS
EPW = E // NW
CH = 80
NCH = 124
TL = EPW - NCH * CH
NP = 10112
RPT = NP // NS


def _proj_body(x_ref, w_ref, o_ref):
    o_ref[...] = jnp.dot(x_ref[...], w_ref[...],
                         preferred_element_type=jnp.float32,
                         precision=lax.Precision.HIGHEST)


def _project(x, w_node):
    return pl.pallas_call(
        _proj_body,
        grid=(5,),
        in_specs=[
            pl.BlockSpec((2000, DN), lambda i: (i, 0)),
            pl.BlockSpec((DN, D), lambda i: (0, 0)),
        ],
        out_specs=pl.BlockSpec((2000, D), lambda i: (i, 0)),
        out_shape=jax.ShapeDtypeStruct((N, D), jnp.float32),
    )(x, w_node)


def _edge_proj_body(at_ref, w_ref, o_ref):
    o_ref[...] = lax.dot_general(
        at_ref[...], w_ref[...], (((0,), (0,)), ((), ())),
        preferred_element_type=jnp.float32,
        precision=lax.Precision.DEFAULT)


def _edge_proj(attr_t8, we8):
    return pl.pallas_call(
        _edge_proj_body,
        grid=(20,),
        in_specs=[
            pl.BlockSpec((8, 16000), lambda i: (0, i)),
            pl.BlockSpec((8, D), lambda i: (0, 0)),
        ],
        out_specs=pl.BlockSpec((16000, D), lambda i: (i, 0)),
        out_shape=jax.ShapeDtypeStruct((E, D), jnp.float32),
    )(attr_t8, we8)


_MESH = plsc.VectorSubcoreMesh(core_axis_name="c", subcore_axis_name="s")


@functools.partial(
    pl.kernel,
    mesh=_MESH,
    out_type=jax.ShapeDtypeStruct((NC, NP, D), jnp.float32),
    scratch_types=[
        pltpu.VMEM((CH,), jnp.int32),
        pltpu.VMEM((CH,), jnp.int32),
        pltpu.VMEM((CH,), jnp.int32),
        pltpu.VMEM((CH,), jnp.int32),
        pltpu.VMEM((CH, D), jnp.float32),
        pltpu.VMEM((CH, D), jnp.float32),
        pltpu.VMEM((CH, D), jnp.float32),
        pltpu.VMEM((CH, D), jnp.float32),
        pltpu.VMEM_SHARED((NP, D), jnp.float32),
        pltpu.SemaphoreType.DMA,
        pltpu.SemaphoreType.DMA,
        pltpu.SemaphoreType.DMA,
        pltpu.SemaphoreType.DMA,
    ],
)
def _sc_agg(h_hbm, p_hbm, src_hbm, dst_hbm, z128_hbm, agg_out,
            src_a, dst_a, src_b, dst_b, rows_a, rows_b, p_a, p_b, acc,
            sem_la, sem_lb, sem_ga, sem_gb):
    cid = lax.axis_index("c")
    sid = lax.axis_index("s")
    wid = cid * NS + sid
    row0 = pl.multiple_of(sid * RPT, 8)
    e0 = pl.multiple_of(wid * EPW, 8)

    def start_loads(c, src_x, dst_x, p_x, sem):
        base = pl.multiple_of(e0 + c * CH, 8)
        pltpu.async_copy(src_hbm.at[pl.ds(base, CH)], src_x, sem)
        pltpu.async_copy(dst_hbm.at[pl.ds(base, CH)], dst_x, sem)
        pltpu.async_copy(p_hbm.at[pl.ds(base, CH)], p_x, sem)

    def wait_loads(src_x, dst_x, p_x, sem):
        pltpu.make_async_copy(src_hbm.at[pl.ds(0, CH)], src_x, sem).wait()
        pltpu.make_async_copy(dst_hbm.at[pl.ds(0, CH)], dst_x, sem).wait()
        pltpu.make_async_copy(p_hbm.at[pl.ds(0, CH)], p_x, sem).wait()

    def scatter(rows_x, p_x, dst_x):
        pltpu.sync_copy(rows_x, acc.at[dst_x], add=True)
        pltpu.sync_copy(p_x, acc.at[dst_x], add=True)

    start_loads(0, src_a, dst_a, p_a, sem_la)
    pltpu.sync_copy(z128_hbm.at[pl.ds(row0, RPT)], acc.at[pl.ds(row0, RPT)])
    plsc.subcore_barrier()

    def body(i, carry):
        wait_loads(src_a, dst_a, p_a, sem_la)
        pltpu.async_copy(h_hbm.at[src_a], rows_a, sem_ga)

        @pl.when(i > 0)
        def _():
            pltpu.make_async_copy(h_hbm.at[src_b], rows_b, sem_gb).wait()
            scatter(rows_b, p_b, dst_b)

        start_loads(2 * i + 1, src_b, dst_b, p_b, sem_lb)
        wait_loads(src_b, dst_b, p_b, sem_lb)
        pltpu.async_copy(h_hbm.at[src_b], rows_b, sem_gb)

        pltpu.make_async_copy(h_hbm.at[src_a], rows_a, sem_ga).wait()
        scatter(rows_a, p_a, dst_a)

        @pl.when(i < NCH // 2 - 1)
        def _():
            start_loads(2 * i + 2, src_a, dst_a, p_a, sem_la)

        return carry

    lax.fori_loop(0, NCH // 2, body, 0)

    pltpu.make_async_copy(h_hbm.at[src_b], rows_b, sem_gb).wait()
    scatter(rows_b, p_b, dst_b)

    tbase = pl.multiple_of(e0 + NCH * CH, 8)
    pltpu.sync_copy(src_hbm.at[pl.ds(tbase, TL)], src_a)
    pltpu.sync_copy(dst_hbm.at[pl.ds(tbase, TL)], dst_a)
    pltpu.async_copy(h_hbm.at[src_a], rows_a, sem_ga).wait()
    pltpu.sync_copy(p_hbm.at[pl.ds(tbase, TL)], p_a)
    scatter(rows_a, p_a, dst_a)

    plsc.subcore_barrier()

    pltpu.sync_copy(acc.at[pl.ds(row0, RPT)],
                    agg_out.at[cid, pl.ds(row0, RPT)])


def _fin_body(h_ref, agg_ref, wo_ref, b_ref, o_ref):
    emb = jnp.maximum(h_ref[...] + agg_ref[0] + agg_ref[1], 0.0)
    logit = jnp.dot(emb, wo_ref[...], preferred_element_type=jnp.float32,
                    precision=lax.Precision.HIGHEST) + b_ref[0, 0]
    o_ref[...] = jax.nn.sigmoid(logit)


def _finish(h, agg, w_out, b2):
    return pl.pallas_call(
        _fin_body,
        grid=(5,),
        in_specs=[
            pl.BlockSpec((2000, D), lambda i: (i, 0)),
            pl.BlockSpec((NC, 2000, D), lambda i: (0, i, 0)),
            pl.BlockSpec((D, 1), lambda i: (0, 0)),
            pl.BlockSpec(memory_space=pltpu.SMEM),
        ],
        out_specs=pl.BlockSpec((2000, 1), lambda i: (i, 0)),
        out_shape=jax.ShapeDtypeStruct((N, 1), jnp.float32),
    )(h, agg, w_out, b2)


def kernel(x, edge_index, edge_attr, W_node, W_edge, W_out, b_out):
    ei = edge_index.astype(jnp.int32)
    src = ei[0]
    dst = ei[1]
    z128 = jnp.zeros((NP, D), jnp.float32)

    attr_t8 = jnp.pad(edge_attr.T, ((0, 3), (0, 0)))
    we8 = jnp.pad(W_edge, ((0, 3), (0, 0)))

    h = _project(x, W_node)
    p = _edge_proj(attr_t8, we8)
    agg = _sc_agg(h, p, src, dst, z128)

    return _finish(h, agg, W_out, b_out.reshape(1, 1)).reshape(N)


# speedup vs baseline: 1.2064x; 1.0368x over previous
import functools

import jax
import jax.numpy as jnp
from jax import lax
from jax.experimental import pallas as pl
from jax.experimental.pallas import tpu as pltpu
from jax.experimental.pallas import tpu_sc as plsc

N = 10000
E = 320000
D = 128
DN = 142
NC = 2
NS = 16
NW = NC * NS
EPW = E // NW
CH = 96
NCH = 104
TL = EPW - NCH * CH
NP = 10112
RPT = NP // NS


def _proj_body(x_ref, w_ref, o_ref):
    o_ref[...] = jnp.dot(x_ref[...], w_ref[...],
                         preferred_element_type=jnp.float32,
                         precision=lax.Precision.HIGHEST)


def _project(x, w_node):
    return pl.pallas_call(
        _proj_body,
        grid=(5,),
        in_specs=[
            pl.BlockSpec((2000, DN), lambda i: (i, 0)),
            pl.BlockSpec((DN, D), lambda i: (0, 0)),
        ],
        out_specs=pl.BlockSpec((2000, D), lambda i: (i, 0)),
        out_shape=jax.ShapeDtypeStruct((N, D), jnp.float32),
    )(x, w_node)


def _edge_proj_body(at_ref, w_ref, o_ref):
    o_ref[...] = lax.dot_general(
        at_ref[...], w_ref[...], (((0,), (0,)), ((), ())),
        preferred_element_type=jnp.float32,
        precision=lax.Precision.DEFAULT)


def _edge_proj(attr_t8, we8):
    return pl.pallas_call(
        _edge_proj_body,
        grid=(20,),
        in_specs=[
            pl.BlockSpec((8, 16000), lambda i: (0, i)),
            pl.BlockSpec((8, D), lambda i: (0, 0)),
        ],
        out_specs=pl.BlockSpec((16000, D), lambda i: (i, 0)),
        out_shape=jax.ShapeDtypeStruct((E, D), jnp.float32),
    )(attr_t8, we8)


_MESH = plsc.VectorSubcoreMesh(core_axis_name="c", subcore_axis_name="s")


@functools.partial(
    pl.kernel,
    mesh=_MESH,
    out_type=jax.ShapeDtypeStruct((NC, NP, D), jnp.float32),
    scratch_types=[
        pltpu.VMEM((CH,), jnp.int32),
        pltpu.VMEM((CH,), jnp.int32),
        pltpu.VMEM((CH,), jnp.int32),
        pltpu.VMEM((CH,), jnp.int32),
        pltpu.VMEM((CH, D), jnp.float32),
        pltpu.VMEM((CH, D), jnp.float32),
        pltpu.VMEM((CH, D), jnp.float32),
        pltpu.VMEM((CH, D), jnp.float32),
        pltpu.VMEM((TL,), jnp.int32),
        pltpu.VMEM((TL,), jnp.int32),
        pltpu.VMEM_SHARED((NP, D), jnp.float32),
        pltpu.SemaphoreType.DMA,
        pltpu.SemaphoreType.DMA,
        pltpu.SemaphoreType.DMA,
        pltpu.SemaphoreType.DMA,
    ],
)
def _sc_agg(h_hbm, p_hbm, src_hbm, dst_hbm, z128_hbm, agg_out,
            src_a, dst_a, src_b, dst_b, rows_a, rows_b, p_a, p_b,
            src_t, dst_t, acc,
            sem_la, sem_lb, sem_ga, sem_gb):
    cid = lax.axis_index("c")
    sid = lax.axis_index("s")
    wid = cid * NS + sid
    row0 = pl.multiple_of(sid * RPT, 8)
    e0 = pl.multiple_of(wid * EPW, 8)

    def start_loads(c, src_x, dst_x, p_x, sem):
        base = pl.multiple_of(e0 + c * CH, 8)
        pltpu.async_copy(src_hbm.at[pl.ds(base, CH)], src_x, sem)
        pltpu.async_copy(dst_hbm.at[pl.ds(base, CH)], dst_x, sem)
        pltpu.async_copy(p_hbm.at[pl.ds(base, CH)], p_x, sem)

    def wait_loads(src_x, dst_x, p_x, sem):
        pltpu.make_async_copy(src_hbm.at[pl.ds(0, CH)], src_x, sem).wait()
        pltpu.make_async_copy(dst_hbm.at[pl.ds(0, CH)], dst_x, sem).wait()
        pltpu.make_async_copy(p_hbm.at[pl.ds(0, CH)], p_x, sem).wait()

    def scatter(rows_x, p_x, dst_x):
        pltpu.sync_copy(rows_x, acc.at[dst_x], add=True)
        pltpu.sync_copy(p_x, acc.at[dst_x], add=True)

    start_loads(0, src_a, dst_a, p_a, sem_la)
    pltpu.sync_copy(z128_hbm.at[pl.ds(row0, RPT)], acc.at[pl.ds(row0, RPT)])
    plsc.subcore_barrier()

    def body(i, carry):
        wait_loads(src_a, dst_a, p_a, sem_la)
        pltpu.async_copy(h_hbm.at[src_a], rows_a, sem_ga)

        @pl.when(i > 0)
        def _():
            pltpu.make_async_copy(h_hbm.at[src_b], rows_b, sem_gb).wait()
            scatter(rows_b, p_b, dst_b)

        start_loads(2 * i + 1, src_b, dst_b, p_b, sem_lb)
        wait_loads(src_b, dst_b, p_b, sem_lb)
        pltpu.async_copy(h_hbm.at[src_b], rows_b, sem_gb)

        pltpu.make_async_copy(h_hbm.at[src_a], rows_a, sem_ga).wait()
        scatter(rows_a, p_a, dst_a)

        @pl.when(i < NCH // 2 - 1)
        def _():
            start_loads(2 * i + 2, src_a, dst_a, p_a, sem_la)

        return carry

    lax.fori_loop(0, NCH // 2, body, 0)

    pltpu.make_async_copy(h_hbm.at[src_b], rows_b, sem_gb).wait()
    scatter(rows_b, p_b, dst_b)

    tbase = pl.multiple_of(e0 + NCH * CH, 8)
    pltpu.sync_copy(src_hbm.at[pl.ds(tbase, TL)], src_t)
    pltpu.sync_copy(dst_hbm.at[pl.ds(tbase, TL)], dst_t)
    pltpu.async_copy(h_hbm.at[src_t], rows_a.at[pl.ds(0, TL)], sem_ga).wait()
    pltpu.sync_copy(p_hbm.at[pl.ds(tbase, TL)], p_a.at[pl.ds(0, TL)])
    pltpu.sync_copy(rows_a.at[pl.ds(0, TL)], acc.at[dst_t], add=True)
    pltpu.sync_copy(p_a.at[pl.ds(0, TL)], acc.at[dst_t], add=True)

    plsc.subcore_barrier()

    pltpu.sync_copy(acc.at[pl.ds(row0, RPT)],
                    agg_out.at[cid, pl.ds(row0, RPT)])


def _fin_body(h_ref, agg_ref, wo_ref, b_ref, o_ref):
    emb = jnp.maximum(h_ref[...] + agg_ref[0] + agg_ref[1], 0.0)
    logit = jnp.dot(emb, wo_ref[...], preferred_element_type=jnp.float32,
                    precision=lax.Precision.HIGHEST) + b_ref[0, 0]
    o_ref[...] = jax.nn.sigmoid(logit)


def _finish(h, agg, w_out, b2):
    return pl.pallas_call(
        _fin_body,
        grid=(5,),
        in_specs=[
            pl.BlockSpec((2000, D), lambda i: (i, 0)),
            pl.BlockSpec((NC, 2000, D), lambda i: (0, i, 0)),
            pl.BlockSpec((D, 1), lambda i: (0, 0)),
            pl.BlockSpec(memory_space=pltpu.SMEM),
        ],
        out_specs=pl.BlockSpec((2000, 1), lambda i: (i, 0)),
        out_shape=jax.ShapeDtypeStruct((N, 1), jnp.float32),
    )(h, agg, w_out, b2)


def kernel(x, edge_index, edge_attr, W_node, W_edge, W_out, b_out):
    ei = edge_index.astype(jnp.int32)
    src = ei[0]
    dst = ei[1]
    z128 = jnp.zeros((NP, D), jnp.float32)

    attr_t8 = jnp.pad(edge_attr.T, ((0, 3), (0, 0)))
    we8 = jnp.pad(W_edge, ((0, 3), (0, 0)))

    h = _project(x, W_node)
    p = _edge_proj(attr_t8, we8)
    agg = _sc_agg(h, p, src, dst, z128)

    return _finish(h, agg, W_out, b_out.reshape(1, 1)).reshape(N)
